# baseline (speedup 1.0000x reference)
import functools
import math

import jax
import jax.numpy as jnp
from jax import lax
from jax.experimental import pallas as pl
from jax.experimental.pallas import tpu as pltpu
from jax.experimental.pallas import tpu_sc as plsc

D_MODEL = 64
SCALE = math.sqrt(D_MODEL)
NUM_CORES = 2
NUM_SUBCORES = 16
NUM_WORKERS = NUM_CORES * NUM_SUBCORES
CHUNK = 160
NBUF = 2
LANES = 16


def _make_kernel(B):
    assert B % (8 * NUM_WORKERS) == 0
    b_per_w = B // NUM_WORKERS
    assert b_per_w % (CHUNK * NBUF) == 0 and CHUNK % LANES == 0
    n_chunks = b_per_w // CHUNK

    mesh = plsc.VectorSubcoreMesh(
        core_axis_name="c",
        subcore_axis_name="s",
        num_cores=NUM_CORES,
        num_subcores=NUM_SUBCORES,
    )

    @functools.partial(
        pl.kernel,
        mesh=mesh,
        compiler_params=pltpu.CompilerParams(needs_layout_passes=False),
        out_type=jax.ShapeDtypeStruct((B, D_MODEL), jnp.float32),
        scratch_types=[
            pltpu.VMEM((b_per_w,), jnp.int32),
        ]
        + [pltpu.VMEM((CHUNK,), jnp.int32)] * NBUF
        + [pltpu.VMEM((CHUNK, 2 * D_MODEL), jnp.float32)] * NBUF
        + [pltpu.VMEM((CHUNK, D_MODEL), jnp.float32)] * NBUF
        + [pltpu.SemaphoreType.DMA] * (2 * NBUF),
    )
    def emb(idx_hbm, table_hbm, out_hbm, idx_all,
            i0, i1, r0, r1, w0, w1, g0, g1, s0, s1):
        sup = (i0, i1)
        rows = (r0, r1)
        outb = (w0, w1)
        gsem = (g0, g1)
        ssem = (s0, s1)
        wid = lax.axis_index("s") * NUM_CORES + lax.axis_index("c")
        base = wid * b_per_w

        pltpu.sync_copy(idx_hbm.at[pl.ds(base, b_per_w)], idx_all)

        def g_copy(b):
            return pltpu.make_async_copy(
                table_hbm.at[sup[b]], rows[b], gsem[b])

        def s_copy(g, b):
            return pltpu.make_async_copy(
                outb[b], out_hbm.at[pl.ds(base + g * CHUNK, CHUNK)], ssem[b])

        def build_sup(g, b):
            goff = g * CHUNK

            @plsc.parallel_loop(0, CHUNK // LANES, step=1, unroll=2)
            def _(k):
                v = idx_all[pl.ds(goff + k * LANES, LANES)]
                sup[b][pl.ds(k * LANES, LANES)] = v >> 1

        def select(g, b):
            goff = g * CHUNK

            @plsc.parallel_loop(0, CHUNK // LANES, step=1, unroll=4)
            def _(k):
                pv = (idx_all[pl.ds(goff + k * LANES, LANES)] & 1) << 6
                for t in range(LANES):
                    i = k * LANES + t
                    poff = pv[t]
                    for j in range(D_MODEL // LANES):
                        v = rows[b][i, pl.ds(poff + j * LANES, LANES)]
                        outb[b][i, pl.ds(j * LANES, LANES)] = v * SCALE

        def process(g, b, prefetch_g=None, sswait_g=None):
            if sswait_g is not None:
                s_copy(sswait_g, 1 - b).wait()
            if prefetch_g is not None:
                build_sup(prefetch_g, 1 - b)
                g_copy(1 - b).start()
            g_copy(b).wait()
            select(g, b)
            s_copy(g, b).start()

        build_sup(0, 0)
        g_copy(0).start()
        process(0, 0, prefetch_g=1)

        def group(i, carry):
            gbase = 1 + i * NBUF
            for j in range(NBUF):
                g = gbase + j
                b = (1 + j) % NBUF
                process(g, b, prefetch_g=g + 1, sswait_g=g - 1)
            return carry

        lax.fori_loop(0, (n_chunks - 2) // NBUF, group, 0)

        process(n_chunks - 1, (n_chunks - 1) % NBUF, sswait_g=n_chunks - 2)
        s_copy(n_chunks - 1, (n_chunks - 1) % NBUF).wait()

    return emb


def kernel(input, table):
    B0, B1 = input.shape
    B = B0 * B1
    idx = input.reshape(B).astype(jnp.int32)
    table_pairs = table.reshape(table.shape[0] // 2, 2 * D_MODEL)
    out = _make_kernel(B)(idx, table_pairs)
    return out.reshape(B0, B1, D_MODEL)


# speedup vs baseline: 1.0944x; 1.0944x over previous
import functools
import math

import jax
import jax.numpy as jnp
from jax import lax
from jax.experimental import pallas as pl
from jax.experimental.pallas import tpu as pltpu
from jax.experimental.pallas import tpu_sc as plsc

D_MODEL = 64
SCALE = math.sqrt(D_MODEL)
NUM_CORES = 2
NUM_SUBCORES = 16
NUM_WORKERS = NUM_CORES * NUM_SUBCORES
CHUNK = 160
NBUF = 2
LANES = 16


def _make_kernel(B):
    assert B % (8 * NUM_WORKERS) == 0
    b_per_w = B // NUM_WORKERS
    assert b_per_w % (CHUNK * NBUF) == 0 and CHUNK % LANES == 0
    n_chunks = b_per_w // CHUNK

    mesh = plsc.VectorSubcoreMesh(
        core_axis_name="c",
        subcore_axis_name="s",
        num_cores=NUM_CORES,
        num_subcores=NUM_SUBCORES,
    )

    @functools.partial(
        pl.kernel,
        mesh=mesh,
        compiler_params=pltpu.CompilerParams(needs_layout_passes=False),
        out_type=jax.ShapeDtypeStruct((B, D_MODEL), jnp.float32),
        scratch_types=[
            pltpu.VMEM((b_per_w,), jnp.int32),
        ]
        + [pltpu.VMEM((CHUNK,), jnp.int32)] * NBUF
        + [pltpu.VMEM((CHUNK, 2 * D_MODEL), jnp.float32)] * NBUF
        + [pltpu.VMEM((CHUNK, D_MODEL), jnp.float32)] * NBUF
        + [pltpu.SemaphoreType.DMA] * (2 * NBUF),
    )
    def emb(idx_hbm, table_hbm, out_hbm, idx_all,
            i0, i1, r0, r1, w0, w1, g0, g1, s0, s1):
        sup = (i0, i1)
        rows = (r0, r1)
        outb = (w0, w1)
        gsem = (g0, g1)
        ssem = (s0, s1)
        wid = lax.axis_index("s") * NUM_CORES + lax.axis_index("c")
        base = wid * b_per_w

        pltpu.sync_copy(idx_hbm.at[pl.ds(base, b_per_w)], idx_all)

        def g_copy(b):
            return pltpu.make_async_copy(
                table_hbm.at[sup[b]], rows[b], gsem[b])

        def s_copy(g, b):
            return pltpu.make_async_copy(
                outb[b], out_hbm.at[pl.ds(base + g * CHUNK, CHUNK)], ssem[b])

        def build_sup(g, b):
            goff = g * CHUNK

            @plsc.parallel_loop(0, CHUNK // LANES, step=1, unroll=2)
            def _(k):
                v = idx_all[pl.ds(goff + k * LANES, LANES)]
                sup[b][pl.ds(k * LANES, LANES)] = v >> 1

        def select(g, b):
            goff = g * CHUNK

            @plsc.parallel_loop(0, CHUNK // LANES, step=1, unroll=2)
            def _(k):
                pv = (idx_all[pl.ds(goff + k * LANES, LANES)] & 1) << 6
                for t in range(LANES):
                    i = k * LANES + t
                    poff = pv[t]
                    for j in range(D_MODEL // LANES):
                        v = rows[b][i, pl.ds(poff + j * LANES, LANES)]
                        outb[b][i, pl.ds(j * LANES, LANES)] = v * SCALE

        def process(g, b, prefetch_g=None, sswait_g=None):
            if sswait_g is not None:
                s_copy(sswait_g, 1 - b).wait()
            if prefetch_g is not None:
                build_sup(prefetch_g, 1 - b)
                g_copy(1 - b).start()
            g_copy(b).wait()
            select(g, b)
            s_copy(g, b).start()

        build_sup(0, 0)
        g_copy(0).start()
        process(0, 0, prefetch_g=1)

        def group(i, carry):
            gbase = 1 + i * NBUF
            for j in range(NBUF):
                g = gbase + j
                b = (1 + j) % NBUF
                process(g, b, prefetch_g=g + 1, sswait_g=g - 1)
            return carry

        lax.fori_loop(0, (n_chunks - 2) // NBUF, group, 0)

        process(n_chunks - 1, (n_chunks - 1) % NBUF, sswait_g=n_chunks - 2)
        s_copy(n_chunks - 1, (n_chunks - 1) % NBUF).wait()

    return emb


def kernel(input, table):
    B0, B1 = input.shape
    B = B0 * B1
    idx = input.reshape(B).astype(jnp.int32)
    table_pairs = table.reshape(table.shape[0] // 2, 2 * D_MODEL)
    out = _make_kernel(B)(idx, table_pairs)
    return out.reshape(B0, B1, D_MODEL)


# speedup vs baseline: 1.1356x; 1.0377x over previous
import functools
import math

import jax
import jax.numpy as jnp
from jax import lax
from jax.experimental import pallas as pl
from jax.experimental.pallas import tpu as pltpu
from jax.experimental.pallas import tpu_sc as plsc

D_MODEL = 64
SCALE = math.sqrt(D_MODEL)
NUM_CORES = 2
NUM_SUBCORES = 16
NUM_WORKERS = NUM_CORES * NUM_SUBCORES
CHUNK = 160
NBUF = 2
LANES = 16


def _make_kernel(B):
    assert B % (8 * NUM_WORKERS) == 0
    b_per_w = B // NUM_WORKERS
    assert b_per_w % (CHUNK * NBUF) == 0 and CHUNK % LANES == 0
    n_chunks = b_per_w // CHUNK

    mesh = plsc.VectorSubcoreMesh(
        core_axis_name="c",
        subcore_axis_name="s",
        num_cores=NUM_CORES,
        num_subcores=NUM_SUBCORES,
    )

    @functools.partial(
        pl.kernel,
        mesh=mesh,
        compiler_params=pltpu.CompilerParams(needs_layout_passes=False),
        out_type=jax.ShapeDtypeStruct((B, D_MODEL), jnp.float32),
        scratch_types=[
            pltpu.VMEM((b_per_w,), jnp.int32),
        ]
        + [pltpu.VMEM((CHUNK,), jnp.int32)] * NBUF
        + [pltpu.VMEM((CHUNK, 2 * D_MODEL), jnp.float32)] * NBUF
        + [pltpu.VMEM((CHUNK, D_MODEL), jnp.float32)] * NBUF
        + [pltpu.SemaphoreType.DMA] * (2 * NBUF),
    )
    def emb(idx_hbm, table_hbm, out_hbm, idx_all,
            i0, i1, r0, r1, w0, w1, g0, g1, s0, s1):
        sup = (i0, i1)
        rows = (r0, r1)
        outb = (w0, w1)
        gsem = (g0, g1)
        ssem = (s0, s1)
        wid = lax.axis_index("s") * NUM_CORES + lax.axis_index("c")
        base = wid * b_per_w

        pltpu.sync_copy(idx_hbm.at[pl.ds(base, b_per_w)], idx_all)

        def g_copy(b):
            return pltpu.make_async_copy(
                table_hbm.at[sup[b]], rows[b], gsem[b])

        def s_copy(g, b):
            return pltpu.make_async_copy(
                outb[b], out_hbm.at[pl.ds(base + g * CHUNK, CHUNK)], ssem[b])

        def build_sup(g, b):
            goff = g * CHUNK

            @plsc.parallel_loop(0, CHUNK // LANES, step=1, unroll=2)
            def _(k):
                v = idx_all[pl.ds(goff + k * LANES, LANES)]
                sup[b][pl.ds(k * LANES, LANES)] = v >> 1

        def select(g, b):
            goff = g * CHUNK

            @plsc.parallel_loop(0, CHUNK // LANES, step=1, unroll=2)
            def _(k):
                pv = idx_all[pl.ds(goff + k * LANES, LANES)] & 1
                for t in range(LANES):
                    i = k * LANES + t
                    splat_t = jnp.full((LANES,), t, jnp.int32)
                    m = pv.at[splat_t].get(mode="promise_in_bounds") == 1
                    for j in range(D_MODEL // LANES):
                        lo = rows[b][i, pl.ds(j * LANES, LANES)]
                        hi = rows[b][i, pl.ds(D_MODEL + j * LANES, LANES)]
                        outb[b][i, pl.ds(j * LANES, LANES)] = (
                            jnp.where(m, hi, lo) * SCALE)

        def process(g, b, prefetch_g=None, sswait_g=None):
            if sswait_g is not None:
                s_copy(sswait_g, 1 - b).wait()
            if prefetch_g is not None:
                build_sup(prefetch_g, 1 - b)
                g_copy(1 - b).start()
            g_copy(b).wait()
            select(g, b)
            s_copy(g, b).start()

        build_sup(0, 0)
        g_copy(0).start()
        process(0, 0, prefetch_g=1)

        def group(i, carry):
            gbase = 1 + i * NBUF
            for j in range(NBUF):
                g = gbase + j
                b = (1 + j) % NBUF
                process(g, b, prefetch_g=g + 1, sswait_g=g - 1)
            return carry

        lax.fori_loop(0, (n_chunks - 2) // NBUF, group, 0)

        process(n_chunks - 1, (n_chunks - 1) % NBUF, sswait_g=n_chunks - 2)
        s_copy(n_chunks - 1, (n_chunks - 1) % NBUF).wait()

    return emb


def kernel(input, table):
    B0, B1 = input.shape
    B = B0 * B1
    idx = input.reshape(B).astype(jnp.int32)
    table_pairs = table.reshape(table.shape[0] // 2, 2 * D_MODEL)
    out = _make_kernel(B)(idx, table_pairs)
    return out.reshape(B0, B1, D_MODEL)


# speedup vs baseline: 1.1583x; 1.0200x over previous
import functools
import math

import jax
import jax.numpy as jnp
from jax import lax
from jax.experimental import pallas as pl
from jax.experimental.pallas import tpu as pltpu
from jax.experimental.pallas import tpu_sc as plsc

D_MODEL = 64
SCALE = math.sqrt(D_MODEL)
NUM_CORES = 2
NUM_SUBCORES = 16
NUM_WORKERS = NUM_CORES * NUM_SUBCORES
CHUNK = 160
NBUF = 2
LANES = 16


def _make_kernel(B):
    assert B % (8 * NUM_WORKERS) == 0
    b_per_w = B // NUM_WORKERS
    assert b_per_w % (CHUNK * NBUF) == 0 and CHUNK % LANES == 0
    n_chunks = b_per_w // CHUNK

    mesh = plsc.VectorSubcoreMesh(
        core_axis_name="c",
        subcore_axis_name="s",
        num_cores=NUM_CORES,
        num_subcores=NUM_SUBCORES,
    )

    @functools.partial(
        pl.kernel,
        mesh=mesh,
        compiler_params=pltpu.CompilerParams(needs_layout_passes=False),
        out_type=jax.ShapeDtypeStruct((B, D_MODEL), jnp.float32),
        scratch_types=[
            pltpu.VMEM((b_per_w,), jnp.int32),
        ]
        + [pltpu.VMEM((CHUNK,), jnp.int32)] * NBUF
        + [pltpu.VMEM((CHUNK, 2 * D_MODEL), jnp.float32)] * NBUF
        + [pltpu.VMEM((CHUNK, D_MODEL), jnp.float32)] * NBUF
        + [pltpu.SemaphoreType.DMA] * (2 * NBUF),
    )
    def emb(idx_hbm, table_hbm, out_hbm, idx_all,
            i0, i1, r0, r1, w0, w1, g0, g1, s0, s1):
        sup = (i0, i1)
        rows = (r0, r1)
        outb = (w0, w1)
        gsem = (g0, g1)
        ssem = (s0, s1)
        wid = lax.axis_index("s") * NUM_CORES + lax.axis_index("c")
        base = wid * b_per_w

        pltpu.sync_copy(idx_hbm.at[pl.ds(base, b_per_w)], idx_all)

        def g_copy(b):
            return pltpu.make_async_copy(
                table_hbm.at[sup[b]], rows[b], gsem[b])

        def s_copy(g, b):
            return pltpu.make_async_copy(
                outb[b], out_hbm.at[pl.ds(base + g * CHUNK, CHUNK)], ssem[b])

        def build_sup(g, b):
            goff = g * CHUNK

            @plsc.parallel_loop(0, CHUNK // LANES, step=1, unroll=2)
            def _(k):
                v = idx_all[pl.ds(goff + k * LANES, LANES)]
                sup[b][pl.ds(k * LANES, LANES)] = v >> 1

        def select(g, b):
            goff = g * CHUNK

            @plsc.parallel_loop(0, CHUNK // LANES, step=1, unroll=2)
            def _(k):
                pv = idx_all[pl.ds(goff + k * LANES, LANES)] & 1
                for t in range(LANES):
                    i = k * LANES + t
                    splat_t = jnp.full((LANES,), t, jnp.int32)
                    m = pv.at[splat_t].get(mode="promise_in_bounds") == 1
                    for j in range(D_MODEL // LANES):
                        lo = rows[b][i, pl.ds(j * LANES, LANES)]
                        hi = rows[b][i, pl.ds(D_MODEL + j * LANES, LANES)]
                        outb[b][i, pl.ds(j * LANES, LANES)] = (
                            jnp.where(m, hi, lo) * SCALE)

        def process(g, b, prefetch_g=None, sswait_g=None):
            if prefetch_g is not None:
                build_sup(prefetch_g, 1 - b)
                g_copy(1 - b).start()
            g_copy(b).wait()
            if sswait_g is not None:
                s_copy(sswait_g, b).wait()
            select(g, b)
            s_copy(g, b).start()

        build_sup(0, 0)
        g_copy(0).start()
        process(0, 0, prefetch_g=1)
        process(1, 1, prefetch_g=2)

        def group(i, carry):
            gbase = 2 + i * NBUF
            for j in range(NBUF):
                g = gbase + j
                b = j % NBUF
                process(g, b, prefetch_g=g + 1, sswait_g=g - 2)
            return carry

        lax.fori_loop(0, (n_chunks - 4) // NBUF, group, 0)

        process(n_chunks - 2, (n_chunks - 2) % NBUF,
                prefetch_g=n_chunks - 1, sswait_g=n_chunks - 4)
        process(n_chunks - 1, (n_chunks - 1) % NBUF,
                sswait_g=n_chunks - 3)
        s_copy(n_chunks - 2, (n_chunks - 2) % NBUF).wait()
        s_copy(n_chunks - 1, (n_chunks - 1) % NBUF).wait()

    return emb


def kernel(input, table):
    B0, B1 = input.shape
    B = B0 * B1
    idx = input.reshape(B).astype(jnp.int32)
    table_pairs = table.reshape(table.shape[0] // 2, 2 * D_MODEL)
    out = _make_kernel(B)(idx, table_pairs)
    return out.reshape(B0, B1, D_MODEL)
